# Initial kernel scaffold; baseline (speedup 1.0000x reference)
import jax
import jax.numpy as jnp
from jax.experimental import pallas as pl


def kernel(x, W1, a1, W2, a2):
    raise NotImplementedError("write your pallas kernel here")


# speedup vs baseline: 5.8656x; 5.8656x over previous
import jax
import jax.numpy as jnp
import numpy as np
from jax.experimental import pallas as pl

T = 32
F = 4
NB = 512
ALPHA = 0.2


def _attention_stage(h, amat):
    Fm = jnp.dot(amat, h, preferred_element_type=jnp.float32)
    f1 = Fm[:T, :]
    f2 = Fm[T:, :]
    e = f1[:, None, :] + f2[None, :, :]
    e = jnp.maximum(e, ALPHA * e)
    m = jnp.max(e, axis=1, keepdims=True)
    p = jnp.exp(e - m)
    denom = jnp.sum(p, axis=1)
    outs = []
    for c in range(F):
        hc = h[c * T:(c + 1) * T, :]
        num = jnp.sum(p * hc[None, :, :], axis=1)
        o = num / denom
        o = jnp.where(o > 0, o, jnp.exp(jnp.minimum(o, 0.0)) - 1.0)
        outs.append(o)
    return jnp.concatenate(outs, axis=0)


def _fused_kernel(x_ref, mw1_ref, a1_ref, mw2_ref, a2_ref, o_ref):
    xb = x_ref[:, :]
    h1 = jnp.dot(mw1_ref[:, :], xb, preferred_element_type=jnp.float32)
    y1 = _attention_stage(h1, a1_ref[:, :])
    h2 = jnp.dot(mw2_ref[:, :], y1, preferred_element_type=jnp.float32)
    o_ref[:, :] = _attention_stage(h2, a2_ref[:, :])


def _build_mats(W1, a1, W2, a2):
    eye = jnp.eye(T, dtype=jnp.float32)
    mw1 = (W1.T[:, None, None, :] * eye[None, :, :, None]).reshape(F * T, T * F)
    mw2 = (W2.T[:, None, :, None] * eye[None, :, None, :]).reshape(F * T, F * T)

    def amat(a):
        top = (a[:F, 0][None, :, None] * eye[:, None, :]).reshape(T, F * T)
        bot = (a[F:, 0][None, :, None] * eye[:, None, :]).reshape(T, F * T)
        return jnp.concatenate([top, bot], axis=0)

    return mw1, amat(a1), mw2, amat(a2)


def kernel(x, W1, a1, W2, a2):
    B, N, Tx, Fx = x.shape
    n_total = B * N
    n_pad = ((n_total + NB - 1) // NB) * NB
    xt = x.reshape(n_total, Tx * Fx)
    if n_pad != n_total:
        xt = jnp.pad(xt, ((0, n_pad - n_total), (0, 0)))
    xt = xt.T

    mw1, a1m, mw2, a2m = _build_mats(W1, a1, W2, a2)
    grid = n_pad // NB
    out = pl.pallas_call(
        _fused_kernel,
        grid=(grid,),
        in_specs=[
            pl.BlockSpec((T * F, NB), lambda i: (0, i)),
            pl.BlockSpec((T * F, T * F), lambda i: (0, 0)),
            pl.BlockSpec((2 * T, T * F), lambda i: (0, 0)),
            pl.BlockSpec((T * F, T * F), lambda i: (0, 0)),
            pl.BlockSpec((2 * T, T * F), lambda i: (0, 0)),
        ],
        out_specs=pl.BlockSpec((T * F, NB), lambda i: (0, i)),
        out_shape=jax.ShapeDtypeStruct((T * F, n_pad), jnp.float32),
    )(xt, mw1, a1m, mw2, a2m)

    out = out[:, :n_total].reshape(F, T, n_total).transpose(2, 1, 0)
    return out.reshape(B, N, Tx, Fx)


# speedup vs baseline: 8.9817x; 1.5313x over previous
import jax
import jax.numpy as jnp
import numpy as np
from jax.experimental import pallas as pl

T = 32
F = 4
NB = 512
ALPHA = 0.2


LOG2E = 1.4426950408889634


def _attention_stage(h, amat):
    Fm = jnp.dot(amat, h, preferred_element_type=jnp.float32)
    f1 = Fm[:T, :]
    f2 = Fm[T:, :]
    mm = jnp.max(f1, axis=0, keepdims=True) + jnp.max(f2, axis=0, keepdims=True)
    M = jnp.maximum(mm, ALPHA * mm) * LOG2E
    f1a = f1 * LOG2E - M
    f2a = f2 * LOG2E
    f1b = f1 * (ALPHA * LOG2E) - M
    f2b = f2 * (ALPHA * LOG2E)
    ea = f2a[:, None, :] + f1a[None, :, :]
    eb = f2b[:, None, :] + f1b[None, :, :]
    p = jnp.exp2(jnp.maximum(ea, eb))
    denom = jnp.sum(p, axis=0)
    inv = 1.0 / denom
    outs = []
    for c in range(F):
        hc = h[c * T:(c + 1) * T, :]
        num = jnp.sum(p * hc[:, None, :], axis=0)
        o = num * inv
        o = jnp.where(o > 0, o, jnp.exp(jnp.minimum(o, 0.0)) - 1.0)
        outs.append(o)
    return jnp.concatenate(outs, axis=0)


def _fused_kernel(x_ref, mw1_ref, a1_ref, mw2_ref, a2_ref, o_ref):
    xb = x_ref[:, :]
    h1 = jnp.dot(mw1_ref[:, :], xb, preferred_element_type=jnp.float32)
    y1 = _attention_stage(h1, a1_ref[:, :])
    h2 = jnp.dot(mw2_ref[:, :], y1, preferred_element_type=jnp.float32)
    o_ref[:, :] = _attention_stage(h2, a2_ref[:, :])


def _build_mats(W1, a1, W2, a2):
    eye = jnp.eye(T, dtype=jnp.float32)
    mw1 = (W1.T[:, None, None, :] * eye[None, :, :, None]).reshape(F * T, T * F)
    mw2 = (W2.T[:, None, :, None] * eye[None, :, None, :]).reshape(F * T, F * T)

    def amat(a):
        top = (a[:F, 0][None, :, None] * eye[:, None, :]).reshape(T, F * T)
        bot = (a[F:, 0][None, :, None] * eye[:, None, :]).reshape(T, F * T)
        return jnp.concatenate([top, bot], axis=0)

    return mw1, amat(a1), mw2, amat(a2)


def kernel(x, W1, a1, W2, a2):
    B, N, Tx, Fx = x.shape
    n_total = B * N
    n_pad = ((n_total + NB - 1) // NB) * NB
    xt = x.reshape(n_total, Tx * Fx)
    if n_pad != n_total:
        xt = jnp.pad(xt, ((0, n_pad - n_total), (0, 0)))
    xt = xt.T

    mw1, a1m, mw2, a2m = _build_mats(W1, a1, W2, a2)
    grid = n_pad // NB
    out = pl.pallas_call(
        _fused_kernel,
        grid=(grid,),
        in_specs=[
            pl.BlockSpec((T * F, NB), lambda i: (0, i)),
            pl.BlockSpec((T * F, T * F), lambda i: (0, 0)),
            pl.BlockSpec((2 * T, T * F), lambda i: (0, 0)),
            pl.BlockSpec((T * F, T * F), lambda i: (0, 0)),
            pl.BlockSpec((2 * T, T * F), lambda i: (0, 0)),
        ],
        out_specs=pl.BlockSpec((T * F, NB), lambda i: (0, i)),
        out_shape=jax.ShapeDtypeStruct((T * F, n_pad), jnp.float32),
    )(xt, mw1, a1m, mw2, a2m)

    out = out[:, :n_total].reshape(F, T, n_total).transpose(2, 1, 0)
    return out.reshape(B, N, Tx, Fx)


# speedup vs baseline: 11.5620x; 1.2873x over previous
import jax
import jax.numpy as jnp
import numpy as np
from jax.experimental import pallas as pl

T = 32
F = 4
NB = 512
ALPHA = 0.2
LOG2E = 1.4426950408889634


def _attention_stage(xin, fmat, hmat):
    Fm = jnp.dot(fmat, xin, preferred_element_type=jnp.float32)
    hrep = jnp.dot(hmat, xin, preferred_element_type=jnp.float32)
    f1 = Fm[:T, :]
    f2 = Fm[T:2 * T, :]
    f2a = Fm[2 * T:2 * T + 8 * T, :].reshape(T, 1, 8, NB)
    f2b = Fm[2 * T + 8 * T:, :].reshape(T, 1, 8, NB)
    mm = jnp.max(f1, axis=0, keepdims=True) + jnp.max(f2, axis=0, keepdims=True)
    mm = mm * LOG2E
    M = jnp.maximum(mm, ALPHA * mm)
    f1a = (f1 * LOG2E - M).reshape(1, F, 8, NB)
    f1b = (f1 * (ALPHA * LOG2E) - M).reshape(1, F, 8, NB)
    p = jnp.exp2(jnp.maximum(f2a + f1a, f2b + f1b))
    denom = jnp.sum(p, axis=0)
    inv = 1.0 / denom
    hb = hrep.reshape(F, T, 8, NB)
    outs = []
    for c in range(F):
        num = jnp.sum(p * hb[c][:, None, :, :], axis=0)
        o = num * inv
        o = jnp.where(o > 0, o, jnp.exp(jnp.minimum(o, 0.0)) - 1.0)
        outs.append(o.reshape(T, NB))
    return jnp.concatenate(outs, axis=0)


def _fused_kernel(x_ref, f1m_ref, h1m_ref, f2m_ref, h2m_ref, o_ref):
    y1 = _attention_stage(x_ref[:, :], f1m_ref[:, :], h1m_ref[:, :])
    o_ref[:, :] = _attention_stage(y1, f2m_ref[:, :], h2m_ref[:, :])


def _build_mats(W1, a1, W2, a2):
    eye = jnp.eye(T, dtype=jnp.float32)
    mw1 = (W1.T[:, None, None, :] * eye[None, :, :, None]).reshape(F * T, T * F)
    mw2 = (W2.T[:, None, :, None] * eye[None, :, None, :]).reshape(F * T, F * T)

    def amat(a):
        top = (a[:F, 0][None, :, None] * eye[:, None, :]).reshape(T, F * T)
        bot = (a[F:, 0][None, :, None] * eye[:, None, :]).reshape(T, F * T)
        return jnp.concatenate([top, bot], axis=0)

    def stage_mats(mw, am):
        aw = jnp.dot(am, mw)
        f2w = aw[T:, :]
        fmat = jnp.concatenate([
            aw,
            jnp.repeat(f2w * LOG2E, 8, axis=0),
            jnp.repeat(f2w * (ALPHA * LOG2E), 8, axis=0),
        ], axis=0)
        hmat = jnp.repeat(mw, 8, axis=0)
        return fmat, hmat

    f1m, h1m = stage_mats(mw1, amat(a1))
    f2m, h2m = stage_mats(mw2, amat(a2))
    return f1m, h1m, f2m, h2m


def kernel(x, W1, a1, W2, a2):
    B, N, Tx, Fx = x.shape
    n_total = B * N
    n_pad = ((n_total + NB - 1) // NB) * NB
    xt = x.reshape(n_total, Tx * Fx)
    if n_pad != n_total:
        xt = jnp.pad(xt, ((0, n_pad - n_total), (0, 0)))
    xt = xt.T

    f1m, h1m, f2m, h2m = _build_mats(W1, a1, W2, a2)
    grid = n_pad // NB
    out = pl.pallas_call(
        _fused_kernel,
        grid=(grid,),
        in_specs=[
            pl.BlockSpec((T * F, NB), lambda i: (0, i)),
            pl.BlockSpec((2 * T + 16 * T, T * F), lambda i: (0, 0)),
            pl.BlockSpec((8 * T * F, T * F), lambda i: (0, 0)),
            pl.BlockSpec((2 * T + 16 * T, T * F), lambda i: (0, 0)),
            pl.BlockSpec((8 * T * F, T * F), lambda i: (0, 0)),
        ],
        out_specs=pl.BlockSpec((T * F, NB), lambda i: (0, i)),
        out_shape=jax.ShapeDtypeStruct((T * F, n_pad), jnp.float32),
    )(xt, f1m, h1m, f2m, h2m)

    out = out[:, :n_total].reshape(F, T, n_total).transpose(2, 1, 0)
    return out.reshape(B, N, Tx, Fx)
